# Initial kernel scaffold; baseline (speedup 1.0000x reference)
import jax
import jax.numpy as jnp
from jax.experimental import pallas as pl


def kernel(x, concepts):
    raise NotImplementedError("write your pallas kernel here")


# speedup vs baseline: 3.7721x; 3.7721x over previous
import jax
import jax.numpy as jnp
from jax.experimental import pallas as pl

_NUM_INTERVENTIONS = 128
_ROW_BLOCK = 2048


def _masked_select_body(mask_ref, x_ref, c_ref, o_ref):
    m = mask_ref[...]
    o_ref[...] = jnp.where(m > 0.5, 1.0 - c_ref[...], x_ref[...])


def kernel(x, concepts):
    batch, dim = x.shape
    idx = jax.random.permutation(jax.random.key(42), dim)[:_NUM_INTERVENTIONS]
    mask = jnp.zeros((1, dim), jnp.float32).at[0, idx].set(1.0)

    rows = min(_ROW_BLOCK, batch)
    grid = (batch // rows,)
    return pl.pallas_call(
        _masked_select_body,
        grid=grid,
        in_specs=[
            pl.BlockSpec((1, dim), lambda i: (0, 0)),
            pl.BlockSpec((rows, dim), lambda i: (i, 0)),
            pl.BlockSpec((rows, dim), lambda i: (i, 0)),
        ],
        out_specs=pl.BlockSpec((rows, dim), lambda i: (i, 0)),
        out_shape=jax.ShapeDtypeStruct((batch, dim), x.dtype),
    )(mask, x, concepts)
